# baseline (speedup 1.0000x reference)
import functools

import jax
import jax.numpy as jnp
from jax import lax
from jax.experimental import pallas as pl
from jax.experimental.pallas import tpu as pltpu
from jax.experimental.pallas import tpu_sc as plsc

N = 50000
E = 1600000
H = 64
G = 512
D1 = 8
HALF = 25000
ACC = 25600
PT = ACC // 16
E_PAD = 1638400
EPT = E_PAD // 16
WB = 64

NC, NS = 2, 16


def _sc_mesh():
    return plsc.VectorSubcoreMesh(
        core_axis_name="c", subcore_axis_name="s",
        num_cores=NC, num_subcores=NS)


def _make_agg(D, K, CPS):
    SUPE = K * CPS
    NSUP = EPT // SUPE

    @functools.partial(
        pl.kernel,
        out_type=jax.ShapeDtypeStruct((NC, ACC, D), jnp.float32),
        mesh=_sc_mesh(),
        compiler_params=pltpu.CompilerParams(use_tc_tiling_on_sc=False),
        scratch_types=[
            pltpu.VMEM((SUPE,), jnp.int32),
            pltpu.VMEM((SUPE,), jnp.int32),
            pltpu.VMEM((SUPE,), jnp.int32),
            pltpu.VMEM((SUPE,), jnp.int32),
            pltpu.VMEM((CPS, K), jnp.int32),
            pltpu.VMEM((CPS, K), jnp.int32),
            pltpu.VMEM((SUPE, D), jnp.float32),
            pltpu.VMEM((SUPE, D), jnp.float32),
            pltpu.VMEM((WB, D), jnp.float32),
            pltpu.VMEM_SHARED((ACC, D), jnp.float32),
            pltpu.SemaphoreType.DMA,
            pltpu.SemaphoreType.DMA,
            pltpu.SemaphoreType.DMA,
            pltpu.SemaphoreType.DMA,
            pltpu.SemaphoreType.DMA,
            pltpu.SemaphoreType.DMA,
        ],
    )
    def agg(table_hbm, src_hbm, dst_hbm, zeros_hbm, out_hbm,
            srcb0, srcb1, dstb0, dstb1, loc0, loc1,
            rows0, rows1, wb_v, acc,
            lsem0, lsem1, gsem0, gsem1, ssem0, ssem1):
        c = lax.axis_index("c")
        s = lax.axis_index("s")
        iota16 = lax.broadcasted_iota(jnp.int32, (16,), 0)
        srcb = (srcb0, srcb1)
        dstb = (dstb0, dstb1)
        locs = (loc0, loc1)
        rows = (rows0, rows1)
        lsem = (lsem0, lsem1)
        gsem = (gsem0, gsem1)
        ssem = (ssem0, ssem1)
        half0 = c * HALF
        ebase = s * EPT

        pltpu.sync_copy(zeros_hbm.at[pl.ds(s * PT, PT)],
                        acc.at[pl.ds(s * PT, PT)])
        plsc.subcore_barrier()

        def fire_loads(g, a):
            sb = ebase + g * SUPE
            pltpu.async_copy(src_hbm.at[pl.ds(sb, SUPE)], srcb[a], lsem[a])
            pltpu.async_copy(dst_hbm.at[pl.ds(sb, SUPE)], dstb[a], lsem[a])

        def wait_loads(a):
            pltpu.make_async_copy(src_hbm.at[pl.ds(0, SUPE)], srcb[a],
                                  lsem[a]).wait()
            pltpu.make_async_copy(dst_hbm.at[pl.ds(0, SUPE)], dstb[a],
                                  lsem[a]).wait()

        def compute_fire_gathers(a):
            for k in range(CPS):
                for t in range(K // 16):
                    o = k * K + t * 16
                    dv = dstb[a][pl.ds(o, 16)]
                    lv = dv - half0
                    inb = (lv >= 0) & (lv < HALF)
                    dummy = HALF + ((s * 16 + o + iota16) & 511)
                    locs[a][k, pl.ds(t * 16, 16)] = jnp.where(inb, lv, dummy)
                pltpu.async_copy(table_hbm.at[srcb[a].at[pl.ds(k * K, K)]],
                                 rows[a].at[pl.ds(k * K, K)], gsem[a])

        def wait_gathers_fire_scatters(a):
            for k in range(CPS):
                pltpu.make_async_copy(table_hbm.at[srcb[a].at[pl.ds(k * K, K)]],
                                      rows[a].at[pl.ds(k * K, K)],
                                      gsem[a]).wait()
                pltpu.async_copy(rows[a].at[pl.ds(k * K, K)],
                                 acc.at[locs[a].at[k]], ssem[a], add=True)

        def wait_scatters(a):
            for k in range(CPS):
                pltpu.make_async_copy(rows[a].at[pl.ds(k * K, K)],
                                      acc.at[locs[a].at[k]], ssem[a]).wait()

        def body(g, a, static):
            a1 = 1 - a
            wait_scatters(a)
            wait_gathers_fire_scatters(a1)
            if static:
                if g < NSUP - 1:
                    fire_loads(g + 1, a1)
            else:
                @pl.when(g < NSUP - 1)
                def _():
                    fire_loads(g + 1, a1)
            wait_loads(a)
            compute_fire_gathers(a)

        fire_loads(0, 0)
        fire_loads(1, 1)
        wait_loads(0)
        compute_fire_gathers(0)
        wait_gathers_fire_scatters(0)
        fire_loads(2, 0)
        wait_loads(1)
        compute_fire_gathers(1)
        body(2, 0, True)
        nxt = 3
        if (NSUP - nxt) % 2 == 1:
            body(3, 1, True)
            nxt = 4

        def pair(u, _):
            for p in range(2):
                g = nxt + 2 * u + p
                body(g, (nxt + p) % 2, False)
            return 0
        lax.fori_loop(0, (NSUP - nxt) // 2, pair, 0)

        last = (NSUP - 1) % 2
        wait_gathers_fire_scatters(last)
        wait_scatters(1 - last)
        wait_scatters(last)
        plsc.subcore_barrier()

        def wb_body(r, _):
            st = s * PT + r * WB
            pltpu.sync_copy(acc.at[pl.ds(st, WB)], wb_v)
            pltpu.sync_copy(wb_v, out_hbm.at[c, pl.ds(st, WB)])
            return 0
        lax.fori_loop(0, PT // WB, wb_body, 0)

    return agg


_agg8 = _make_agg(D1, 128, 2)
_agg64 = _make_agg(H, 128, 1)


KP = 80
_PCH = N // KP
_PPW = -(-_PCH // (NC * NS))
_GPT = G // NS


@functools.partial(
    pl.kernel,
    out_type=(jax.ShapeDtypeStruct((NC, G, H), jnp.float32),
              jax.ShapeDtypeStruct((NC, G, 8), jnp.float32)),
    mesh=_sc_mesh(),
    compiler_params=pltpu.CompilerParams(use_tc_tiling_on_sc=False),
    scratch_types=[
        pltpu.VMEM((KP,), jnp.int32),
        pltpu.VMEM((KP, H), jnp.float32),
        pltpu.VMEM((KP, 8), jnp.float32),
        pltpu.VMEM((_GPT, H), jnp.float32),
        pltpu.VMEM((_GPT, 8), jnp.float32),
        pltpu.VMEM_SHARED((G, H), jnp.float32),
        pltpu.VMEM_SHARED((G, 8), jnp.float32),
    ],
)
def _pool(h2_hbm, batch_hbm, zs_hbm, zc_hbm, ones_hbm, out_s, out_c,
          b_v, rows_v, ones_v, wbs_v, wbc_v, acc_s, acc_c):
    c = lax.axis_index("c")
    s = lax.axis_index("s")
    wid = s * NC + c

    pltpu.sync_copy(ones_hbm, ones_v)
    pltpu.sync_copy(zs_hbm.at[pl.ds(s * _GPT, _GPT)],
                    acc_s.at[pl.ds(s * _GPT, _GPT)])
    pltpu.sync_copy(zc_hbm.at[pl.ds(s * _GPT, _GPT)],
                    acc_c.at[pl.ds(s * _GPT, _GPT)])
    plsc.subcore_barrier()

    def body(i, _):
        ch = wid * _PPW + i

        @pl.when(ch < _PCH)
        def _():
            nb = ch * KP
            pltpu.sync_copy(batch_hbm.at[pl.ds(nb, KP)], b_v)
            pltpu.sync_copy(h2_hbm.at[pl.ds(nb, KP)], rows_v)
            pltpu.sync_copy(rows_v, acc_s.at[b_v], add=True)
            pltpu.sync_copy(ones_v, acc_c.at[b_v], add=True)
        return 0
    lax.fori_loop(0, _PPW, body, 0)
    plsc.subcore_barrier()

    pltpu.sync_copy(acc_s.at[pl.ds(s * _GPT, _GPT)], wbs_v)
    pltpu.sync_copy(wbs_v, out_s.at[c, pl.ds(s * _GPT, _GPT)])
    pltpu.sync_copy(acc_c.at[pl.ds(s * _GPT, _GPT)], wbc_v)
    pltpu.sync_copy(wbc_v, out_c.at[c, pl.ds(s * _GPT, _GPT)])


_R = 1000


def _mlp_body(x_ref, a_ref, w1_ref, b1_ref, w2_ref, b2_ref, o_ref):
    t = jnp.dot(x_ref[...] + a_ref[0], w1_ref[...],
                preferred_element_type=jnp.float32) + b1_ref[...]
    t = jnp.maximum(t, 0.0)
    h = jnp.dot(t, w2_ref[...], preferred_element_type=jnp.float32) + b2_ref[...]
    o_ref[...] = jnp.maximum(h, 0.0)


def _mlp(x, agg, w1, b1, w2, b2, din):
    return pl.pallas_call(
        _mlp_body,
        grid=(N // _R,),
        in_specs=[
            pl.BlockSpec((_R, din), lambda i: (i, 0)),
            pl.BlockSpec((1, _R, din), lambda i: (i // (HALF // _R),
                                                  i % (HALF // _R), 0)),
            pl.BlockSpec((din, H), lambda i: (0, 0)),
            pl.BlockSpec((1, H), lambda i: (0, 0)),
            pl.BlockSpec((H, H), lambda i: (0, 0)),
            pl.BlockSpec((1, H), lambda i: (0, 0)),
        ],
        out_specs=pl.BlockSpec((_R, H), lambda i: (i, 0)),
        out_shape=jax.ShapeDtypeStruct((N, H), jnp.float32),
    )(x, agg, w1, b1, w2, b2)


def _final_body(s_ref, c_ref, wc_ref, bc_ref, o_ref):
    sums = s_ref[0] + s_ref[1]
    cnt = jnp.maximum(c_ref[0][:, 0:1] + c_ref[1][:, 0:1], 1.0)
    pooled = sums / cnt
    o_ref[...] = jnp.dot(pooled, wc_ref[...],
                         preferred_element_type=jnp.float32) + bc_ref[...]


def kernel(x, edge_index, batch, W1, b1, W2, b2, W3, b3, W4, b4, Wc, bc):
    xp = jnp.pad(x, ((0, 0), (0, D1 - x.shape[1])))
    w1p = jnp.pad(W1, ((0, D1 - W1.shape[0]), (0, 0)))
    src = jnp.pad(edge_index[0].astype(jnp.int32), (0, E_PAD - E))
    dst = jnp.pad(edge_index[1].astype(jnp.int32), (0, E_PAD - E),
                  constant_values=N)
    batch = batch.astype(jnp.int32)
    z8 = jnp.zeros((ACC, D1), jnp.float32)
    z64 = jnp.zeros((ACC, H), jnp.float32)

    agg1 = _agg8(xp, src, dst, z8)
    h = _mlp(xp, agg1, w1p, b1.reshape(1, H), W2, b2.reshape(1, H), D1)
    agg2 = _agg64(h, src, dst, z64)
    h2 = _mlp(h, agg2, W3, b3.reshape(1, H), W4, b4.reshape(1, H), H)
    psums, pcnt = _pool(h2, batch, z64[:G], z8[:G], jnp.ones((KP, 8), jnp.float32))

    return pl.pallas_call(
        _final_body,
        in_specs=[
            pl.BlockSpec((NC, G, H), lambda: (0, 0, 0)),
            pl.BlockSpec((NC, G, 8), lambda: (0, 0, 0)),
            pl.BlockSpec((H, 2), lambda: (0, 0)),
            pl.BlockSpec((1, 2), lambda: (0, 0)),
        ],
        out_specs=pl.BlockSpec((G, 2), lambda: (0, 0)),
        out_shape=jax.ShapeDtypeStruct((G, 2), jnp.float32),
    )(psums, pcnt, Wc, bc.reshape(1, 2))


# speedup vs baseline: 1.0307x; 1.0307x over previous
import functools

import jax
import jax.numpy as jnp
from jax import lax
from jax.experimental import pallas as pl
from jax.experimental.pallas import tpu as pltpu
from jax.experimental.pallas import tpu_sc as plsc

N = 50000
E = 1600000
H = 64
G = 512
D1 = 8
HALF = 25000
ACC = 25600
PT = ACC // 16
E_PAD = 1638400
EPT = E_PAD // 16
WB = 64

NC, NS = 2, 16


def _sc_mesh():
    return plsc.VectorSubcoreMesh(
        core_axis_name="c", subcore_axis_name="s",
        num_cores=NC, num_subcores=NS)


def _make_agg(D, K, CPS):
    SUPE = K * CPS
    NSUP = EPT // SUPE

    @functools.partial(
        pl.kernel,
        out_type=jax.ShapeDtypeStruct((NC, ACC, D), jnp.float32),
        mesh=_sc_mesh(),
        compiler_params=pltpu.CompilerParams(use_tc_tiling_on_sc=False),
        scratch_types=[
            pltpu.VMEM((SUPE,), jnp.int32),
            pltpu.VMEM((SUPE,), jnp.int32),
            pltpu.VMEM((SUPE,), jnp.int32),
            pltpu.VMEM((SUPE,), jnp.int32),
            pltpu.VMEM((CPS, K), jnp.int32),
            pltpu.VMEM((CPS, K), jnp.int32),
            pltpu.VMEM((SUPE, D), jnp.float32),
            pltpu.VMEM((SUPE, D), jnp.float32),
            pltpu.VMEM((WB, D), jnp.float32),
            pltpu.VMEM_SHARED((ACC, D), jnp.float32),
            pltpu.SemaphoreType.DMA,
            pltpu.SemaphoreType.DMA,
            pltpu.SemaphoreType.DMA,
            pltpu.SemaphoreType.DMA,
            pltpu.SemaphoreType.DMA,
            pltpu.SemaphoreType.DMA,
        ],
    )
    def agg(table_hbm, src_hbm, dst_hbm, zeros_hbm, out_hbm,
            srcb0, srcb1, dstb0, dstb1, loc0, loc1,
            rows0, rows1, wb_v, acc,
            lsem0, lsem1, gsem0, gsem1, ssem0, ssem1):
        c = lax.axis_index("c")
        s = lax.axis_index("s")
        iota16 = lax.broadcasted_iota(jnp.int32, (16,), 0)
        srcb = (srcb0, srcb1)
        dstb = (dstb0, dstb1)
        locs = (loc0, loc1)
        rows = (rows0, rows1)
        lsem = (lsem0, lsem1)
        gsem = (gsem0, gsem1)
        ssem = (ssem0, ssem1)
        half0 = c * HALF
        ebase = s * EPT

        pltpu.sync_copy(zeros_hbm.at[pl.ds(s * PT, PT)],
                        acc.at[pl.ds(s * PT, PT)])
        plsc.subcore_barrier()

        def fire_loads(g, a):
            sb = ebase + g * SUPE
            pltpu.async_copy(src_hbm.at[pl.ds(sb, SUPE)], srcb[a], lsem[a])
            pltpu.async_copy(dst_hbm.at[pl.ds(sb, SUPE)], dstb[a], lsem[a])

        def wait_loads(a):
            pltpu.make_async_copy(src_hbm.at[pl.ds(0, SUPE)], srcb[a],
                                  lsem[a]).wait()
            pltpu.make_async_copy(dst_hbm.at[pl.ds(0, SUPE)], dstb[a],
                                  lsem[a]).wait()

        def compute_fire_gathers(a):
            for k in range(CPS):
                for t in range(K // 16):
                    o = k * K + t * 16
                    dv = dstb[a][pl.ds(o, 16)]
                    lv = dv - half0
                    inb = (lv >= 0) & (lv < HALF)
                    dummy = HALF + ((s * 16 + o + iota16) & 511)
                    locs[a][k, pl.ds(t * 16, 16)] = jnp.where(inb, lv, dummy)
                pltpu.async_copy(table_hbm.at[srcb[a].at[pl.ds(k * K, K)]],
                                 rows[a].at[pl.ds(k * K, K)], gsem[a])

        def wait_gathers_fire_scatters(a):
            for k in range(CPS):
                pltpu.make_async_copy(table_hbm.at[srcb[a].at[pl.ds(k * K, K)]],
                                      rows[a].at[pl.ds(k * K, K)],
                                      gsem[a]).wait()
                pltpu.async_copy(rows[a].at[pl.ds(k * K, K)],
                                 acc.at[locs[a].at[k]], ssem[a], add=True)

        def wait_scatters(a):
            for k in range(CPS):
                pltpu.make_async_copy(rows[a].at[pl.ds(k * K, K)],
                                      acc.at[locs[a].at[k]], ssem[a]).wait()

        def body(g, a, static):
            a1 = 1 - a
            wait_scatters(a)
            wait_gathers_fire_scatters(a1)
            if static:
                if g < NSUP - 1:
                    fire_loads(g + 1, a1)
            else:
                @pl.when(g < NSUP - 1)
                def _():
                    fire_loads(g + 1, a1)
            wait_loads(a)
            compute_fire_gathers(a)

        fire_loads(0, 0)
        fire_loads(1, 1)
        wait_loads(0)
        compute_fire_gathers(0)
        wait_gathers_fire_scatters(0)
        fire_loads(2, 0)
        wait_loads(1)
        compute_fire_gathers(1)
        body(2, 0, True)
        nxt = 3
        if (NSUP - nxt) % 2 == 1:
            body(3, 1, True)
            nxt = 4

        def pair(u, _):
            for p in range(2):
                g = nxt + 2 * u + p
                body(g, (nxt + p) % 2, False)
            return 0
        lax.fori_loop(0, (NSUP - nxt) // 2, pair, 0)

        last = (NSUP - 1) % 2
        wait_gathers_fire_scatters(last)
        wait_scatters(1 - last)
        wait_scatters(last)
        plsc.subcore_barrier()

        def wb_body(r, _):
            st = s * PT + r * WB
            pltpu.sync_copy(acc.at[pl.ds(st, WB)], wb_v)
            pltpu.sync_copy(wb_v, out_hbm.at[c, pl.ds(st, WB)])
            return 0
        lax.fori_loop(0, PT // WB, wb_body, 0)

    return agg


_agg8 = _make_agg(D1, 80, 4)
_agg64 = _make_agg(H, 80, 2)


KP = 80
_PCH = N // KP
_PPW = -(-_PCH // (NC * NS))
_GPT = G // NS


@functools.partial(
    pl.kernel,
    out_type=(jax.ShapeDtypeStruct((NC, G, H), jnp.float32),
              jax.ShapeDtypeStruct((NC, G, 8), jnp.float32)),
    mesh=_sc_mesh(),
    compiler_params=pltpu.CompilerParams(use_tc_tiling_on_sc=False),
    scratch_types=[
        pltpu.VMEM((KP,), jnp.int32),
        pltpu.VMEM((KP, H), jnp.float32),
        pltpu.VMEM((KP, 8), jnp.float32),
        pltpu.VMEM((_GPT, H), jnp.float32),
        pltpu.VMEM((_GPT, 8), jnp.float32),
        pltpu.VMEM_SHARED((G, H), jnp.float32),
        pltpu.VMEM_SHARED((G, 8), jnp.float32),
    ],
)
def _pool(h2_hbm, batch_hbm, zs_hbm, zc_hbm, ones_hbm, out_s, out_c,
          b_v, rows_v, ones_v, wbs_v, wbc_v, acc_s, acc_c):
    c = lax.axis_index("c")
    s = lax.axis_index("s")
    wid = s * NC + c

    pltpu.sync_copy(ones_hbm, ones_v)
    pltpu.sync_copy(zs_hbm.at[pl.ds(s * _GPT, _GPT)],
                    acc_s.at[pl.ds(s * _GPT, _GPT)])
    pltpu.sync_copy(zc_hbm.at[pl.ds(s * _GPT, _GPT)],
                    acc_c.at[pl.ds(s * _GPT, _GPT)])
    plsc.subcore_barrier()

    def body(i, _):
        ch = wid * _PPW + i

        @pl.when(ch < _PCH)
        def _():
            nb = ch * KP
            pltpu.sync_copy(batch_hbm.at[pl.ds(nb, KP)], b_v)
            pltpu.sync_copy(h2_hbm.at[pl.ds(nb, KP)], rows_v)
            pltpu.sync_copy(rows_v, acc_s.at[b_v], add=True)
            pltpu.sync_copy(ones_v, acc_c.at[b_v], add=True)
        return 0
    lax.fori_loop(0, _PPW, body, 0)
    plsc.subcore_barrier()

    pltpu.sync_copy(acc_s.at[pl.ds(s * _GPT, _GPT)], wbs_v)
    pltpu.sync_copy(wbs_v, out_s.at[c, pl.ds(s * _GPT, _GPT)])
    pltpu.sync_copy(acc_c.at[pl.ds(s * _GPT, _GPT)], wbc_v)
    pltpu.sync_copy(wbc_v, out_c.at[c, pl.ds(s * _GPT, _GPT)])


_R = 1000


def _mlp_body(x_ref, a_ref, w1_ref, b1_ref, w2_ref, b2_ref, o_ref):
    t = jnp.dot(x_ref[...] + a_ref[0], w1_ref[...],
                preferred_element_type=jnp.float32) + b1_ref[...]
    t = jnp.maximum(t, 0.0)
    h = jnp.dot(t, w2_ref[...], preferred_element_type=jnp.float32) + b2_ref[...]
    o_ref[...] = jnp.maximum(h, 0.0)


def _mlp(x, agg, w1, b1, w2, b2, din):
    return pl.pallas_call(
        _mlp_body,
        grid=(N // _R,),
        in_specs=[
            pl.BlockSpec((_R, din), lambda i: (i, 0)),
            pl.BlockSpec((1, _R, din), lambda i: (i // (HALF // _R),
                                                  i % (HALF // _R), 0)),
            pl.BlockSpec((din, H), lambda i: (0, 0)),
            pl.BlockSpec((1, H), lambda i: (0, 0)),
            pl.BlockSpec((H, H), lambda i: (0, 0)),
            pl.BlockSpec((1, H), lambda i: (0, 0)),
        ],
        out_specs=pl.BlockSpec((_R, H), lambda i: (i, 0)),
        out_shape=jax.ShapeDtypeStruct((N, H), jnp.float32),
    )(x, agg, w1, b1, w2, b2)


def _final_body(s_ref, c_ref, wc_ref, bc_ref, o_ref):
    sums = s_ref[0] + s_ref[1]
    cnt = jnp.maximum(c_ref[0][:, 0:1] + c_ref[1][:, 0:1], 1.0)
    pooled = sums / cnt
    o_ref[...] = jnp.dot(pooled, wc_ref[...],
                         preferred_element_type=jnp.float32) + bc_ref[...]


def kernel(x, edge_index, batch, W1, b1, W2, b2, W3, b3, W4, b4, Wc, bc):
    xp = jnp.pad(x, ((0, 0), (0, D1 - x.shape[1])))
    w1p = jnp.pad(W1, ((0, D1 - W1.shape[0]), (0, 0)))
    src = jnp.pad(edge_index[0].astype(jnp.int32), (0, E_PAD - E))
    dst = jnp.pad(edge_index[1].astype(jnp.int32), (0, E_PAD - E),
                  constant_values=N)
    batch = batch.astype(jnp.int32)
    z8 = jnp.zeros((ACC, D1), jnp.float32)
    z64 = jnp.zeros((ACC, H), jnp.float32)

    agg1 = _agg8(xp, src, dst, z8)
    h = _mlp(xp, agg1, w1p, b1.reshape(1, H), W2, b2.reshape(1, H), D1)
    agg2 = _agg64(h, src, dst, z64)
    h2 = _mlp(h, agg2, W3, b3.reshape(1, H), W4, b4.reshape(1, H), H)
    psums, pcnt = _pool(h2, batch, z64[:G], z8[:G], jnp.ones((KP, 8), jnp.float32))

    return pl.pallas_call(
        _final_body,
        in_specs=[
            pl.BlockSpec((NC, G, H), lambda: (0, 0, 0)),
            pl.BlockSpec((NC, G, 8), lambda: (0, 0, 0)),
            pl.BlockSpec((H, 2), lambda: (0, 0)),
            pl.BlockSpec((1, 2), lambda: (0, 0)),
        ],
        out_specs=pl.BlockSpec((G, 2), lambda: (0, 0)),
        out_shape=jax.ShapeDtypeStruct((G, 2), jnp.float32),
    )(psums, pcnt, Wc, bc.reshape(1, 2))


# speedup vs baseline: 2.3065x; 2.2378x over previous
import functools

import jax
import jax.numpy as jnp
from jax import lax
from jax.experimental import pallas as pl
from jax.experimental.pallas import tpu as pltpu
from jax.experimental.pallas import tpu_sc as plsc

N = 50000
E = 1600000
H = 64
G = 512
D1 = 8
HALF = 25000
ACC = 25600
PT = ACC // 16
EPT = E // 16
WB = 64

NC, NS = 2, 16


def _sc_mesh():
    return plsc.VectorSubcoreMesh(
        core_axis_name="c", subcore_axis_name="s",
        num_cores=NC, num_subcores=NS)


def _make_agg(D, K, CPS):
    SUPE = K * CPS
    NSUP = EPT // SUPE

    @functools.partial(
        pl.kernel,
        out_type=jax.ShapeDtypeStruct((NC, ACC, D), jnp.float32),
        mesh=_sc_mesh(),
        compiler_params=pltpu.CompilerParams(use_tc_tiling_on_sc=False),
        scratch_types=[
            pltpu.VMEM((SUPE,), jnp.int32),
            pltpu.VMEM((SUPE,), jnp.int32),
            pltpu.VMEM((SUPE,), jnp.int32),
            pltpu.VMEM((SUPE,), jnp.int32),
            pltpu.VMEM((CPS, K), jnp.int32),
            pltpu.VMEM((CPS, K), jnp.int32),
            pltpu.VMEM((SUPE, D), jnp.float32),
            pltpu.VMEM((SUPE, D), jnp.float32),
            pltpu.VMEM((WB, D), jnp.float32),
            pltpu.VMEM_SHARED((ACC, D), jnp.float32),
            pltpu.SemaphoreType.DMA,
            pltpu.SemaphoreType.DMA,
            pltpu.SemaphoreType.DMA,
            pltpu.SemaphoreType.DMA,
            pltpu.SemaphoreType.DMA,
            pltpu.SemaphoreType.DMA,
        ],
    )
    def agg(table_hbm, src_hbm, dst_hbm, zeros_hbm, out_hbm,
            srcb0, srcb1, dstb0, dstb1, loc0, loc1,
            rows0, rows1, wb_v, acc,
            lsem0, lsem1, gsem0, gsem1, ssem0, ssem1):
        c = lax.axis_index("c")
        s = lax.axis_index("s")
        iota16 = lax.broadcasted_iota(jnp.int32, (16,), 0)
        srcb = (srcb0, srcb1)
        dstb = (dstb0, dstb1)
        locs = (loc0, loc1)
        rows = (rows0, rows1)
        lsem = (lsem0, lsem1)
        gsem = (gsem0, gsem1)
        ssem = (ssem0, ssem1)
        half0 = c * HALF
        ebase = s * EPT

        pltpu.sync_copy(zeros_hbm.at[pl.ds(s * PT, PT)],
                        acc.at[pl.ds(s * PT, PT)])
        plsc.subcore_barrier()

        def fire_loads(g, a):
            sb = ebase + g * SUPE
            pltpu.async_copy(src_hbm.at[pl.ds(sb, SUPE)], srcb[a], lsem[a])
            pltpu.async_copy(dst_hbm.at[pl.ds(sb, SUPE)], dstb[a], lsem[a])

        def wait_loads(a):
            pltpu.make_async_copy(src_hbm.at[pl.ds(0, SUPE)], srcb[a],
                                  lsem[a]).wait()
            pltpu.make_async_copy(dst_hbm.at[pl.ds(0, SUPE)], dstb[a],
                                  lsem[a]).wait()

        def compute_fire_gathers(a):
            for k in range(CPS):
                for t in range(K // 16):
                    o = k * K + t * 16
                    dv = dstb[a][pl.ds(o, 16)]
                    lv = dv - half0
                    inb = (lv >= 0) & (lv < HALF)
                    dummy = HALF + ((s * 16 + o + iota16) & 511)
                    locs[a][k, pl.ds(t * 16, 16)] = jnp.where(inb, lv, dummy)
                pltpu.async_copy(table_hbm.at[srcb[a].at[pl.ds(k * K, K)]],
                                 rows[a].at[pl.ds(k * K, K)], gsem[a])

        def wait_gathers_fire_scatters(a):
            for k in range(CPS):
                pltpu.make_async_copy(table_hbm.at[srcb[a].at[pl.ds(k * K, K)]],
                                      rows[a].at[pl.ds(k * K, K)],
                                      gsem[a]).wait()
                pltpu.async_copy(rows[a].at[pl.ds(k * K, K)],
                                 acc.at[locs[a].at[k]], ssem[a], add=True)

        def wait_scatters(a):
            for k in range(CPS):
                pltpu.make_async_copy(rows[a].at[pl.ds(k * K, K)],
                                      acc.at[locs[a].at[k]], ssem[a]).wait()

        def body(g, a, static):
            a1 = 1 - a
            wait_scatters(a)
            wait_gathers_fire_scatters(a1)
            if static:
                if g < NSUP - 1:
                    fire_loads(g + 1, a1)
            else:
                @pl.when(g < NSUP - 1)
                def _():
                    fire_loads(g + 1, a1)
            wait_loads(a)
            compute_fire_gathers(a)

        fire_loads(0, 0)
        fire_loads(1, 1)
        wait_loads(0)
        compute_fire_gathers(0)
        wait_gathers_fire_scatters(0)
        fire_loads(2, 0)
        wait_loads(1)
        compute_fire_gathers(1)
        body(2, 0, True)
        nxt = 3
        if (NSUP - nxt) % 2 == 1:
            body(3, 1, True)
            nxt = 4

        def pair(u, _):
            for p in range(2):
                g = nxt + 2 * u + p
                body(g, (nxt + p) % 2, False)
            return 0
        lax.fori_loop(0, (NSUP - nxt) // 2, pair, 0)

        last = (NSUP - 1) % 2
        wait_gathers_fire_scatters(last)
        wait_scatters(1 - last)
        wait_scatters(last)
        plsc.subcore_barrier()

        def wb_body(r, _):
            st = s * PT + r * WB
            pltpu.sync_copy(acc.at[pl.ds(st, WB)], wb_v)
            pltpu.sync_copy(wb_v, out_hbm.at[c, pl.ds(st, WB)])
            return 0
        lax.fori_loop(0, PT // WB, wb_body, 0)

    return agg


_agg8 = _make_agg(D1, 80, 5)
_agg64 = _make_agg(H, 80, 2)


KP = 80
_PCH = N // KP
_PPW = -(-_PCH // (NC * NS))
_GPT = G // NS


@functools.partial(
    pl.kernel,
    out_type=(jax.ShapeDtypeStruct((NC, G, H), jnp.float32),
              jax.ShapeDtypeStruct((NC, G, 8), jnp.float32)),
    mesh=_sc_mesh(),
    compiler_params=pltpu.CompilerParams(use_tc_tiling_on_sc=False),
    scratch_types=[
        pltpu.VMEM((KP,), jnp.int32),
        pltpu.VMEM((KP, H), jnp.float32),
        pltpu.VMEM((KP, 8), jnp.float32),
        pltpu.VMEM((_GPT, H), jnp.float32),
        pltpu.VMEM((_GPT, 8), jnp.float32),
        pltpu.VMEM_SHARED((G, H), jnp.float32),
        pltpu.VMEM_SHARED((G, 8), jnp.float32),
    ],
)
def _pool(h2_hbm, batch_hbm, zs_hbm, zc_hbm, ones_hbm, out_s, out_c,
          b_v, rows_v, ones_v, wbs_v, wbc_v, acc_s, acc_c):
    c = lax.axis_index("c")
    s = lax.axis_index("s")
    wid = s * NC + c

    pltpu.sync_copy(ones_hbm, ones_v)
    pltpu.sync_copy(zs_hbm.at[pl.ds(s * _GPT, _GPT)],
                    acc_s.at[pl.ds(s * _GPT, _GPT)])
    pltpu.sync_copy(zc_hbm.at[pl.ds(s * _GPT, _GPT)],
                    acc_c.at[pl.ds(s * _GPT, _GPT)])
    plsc.subcore_barrier()

    def body(i, _):
        ch = wid * _PPW + i

        @pl.when(ch < _PCH)
        def _():
            nb = ch * KP
            pltpu.sync_copy(batch_hbm.at[pl.ds(nb, KP)], b_v)
            pltpu.sync_copy(h2_hbm.at[pl.ds(nb, KP)], rows_v)
            pltpu.sync_copy(rows_v, acc_s.at[b_v], add=True)
            pltpu.sync_copy(ones_v, acc_c.at[b_v], add=True)
        return 0
    lax.fori_loop(0, _PPW, body, 0)
    plsc.subcore_barrier()

    pltpu.sync_copy(acc_s.at[pl.ds(s * _GPT, _GPT)], wbs_v)
    pltpu.sync_copy(wbs_v, out_s.at[c, pl.ds(s * _GPT, _GPT)])
    pltpu.sync_copy(acc_c.at[pl.ds(s * _GPT, _GPT)], wbc_v)
    pltpu.sync_copy(wbc_v, out_c.at[c, pl.ds(s * _GPT, _GPT)])


_R = 1000


def _mlp_body(x_ref, a_ref, w1_ref, b1_ref, w2_ref, b2_ref, o_ref):
    t = jnp.dot(x_ref[...] + a_ref[0], w1_ref[...],
                preferred_element_type=jnp.float32) + b1_ref[...]
    t = jnp.maximum(t, 0.0)
    h = jnp.dot(t, w2_ref[...], preferred_element_type=jnp.float32) + b2_ref[...]
    o_ref[...] = jnp.maximum(h, 0.0)


def _mlp(x, agg, w1, b1, w2, b2, din):
    return pl.pallas_call(
        _mlp_body,
        grid=(N // _R,),
        in_specs=[
            pl.BlockSpec((_R, din), lambda i: (i, 0)),
            pl.BlockSpec((1, _R, din), lambda i: (i // (HALF // _R),
                                                  i % (HALF // _R), 0)),
            pl.BlockSpec((din, H), lambda i: (0, 0)),
            pl.BlockSpec((1, H), lambda i: (0, 0)),
            pl.BlockSpec((H, H), lambda i: (0, 0)),
            pl.BlockSpec((1, H), lambda i: (0, 0)),
        ],
        out_specs=pl.BlockSpec((_R, H), lambda i: (i, 0)),
        out_shape=jax.ShapeDtypeStruct((N, H), jnp.float32),
    )(x, agg, w1, b1, w2, b2)


def _final_body(s_ref, c_ref, wc_ref, bc_ref, o_ref):
    sums = s_ref[0] + s_ref[1]
    cnt = jnp.maximum(c_ref[0][:, 0:1] + c_ref[1][:, 0:1], 1.0)
    pooled = sums / cnt
    o_ref[...] = jnp.dot(pooled, wc_ref[...],
                         preferred_element_type=jnp.float32) + bc_ref[...]


def kernel(x, edge_index, batch, W1, b1, W2, b2, W3, b3, W4, b4, Wc, bc):
    xp = jnp.pad(x, ((0, 0), (0, D1 - x.shape[1])))
    w1p = jnp.pad(W1, ((0, D1 - W1.shape[0]), (0, 0)))
    src = edge_index[0].astype(jnp.int32)
    dst = edge_index[1].astype(jnp.int32)
    batch = batch.astype(jnp.int32)
    z8 = jnp.zeros((ACC, D1), jnp.float32)
    z64 = jnp.zeros((ACC, H), jnp.float32)

    agg1 = _agg8(xp, src, dst, z8)
    h = _mlp(xp, agg1, w1p, b1.reshape(1, H), W2, b2.reshape(1, H), D1)
    agg2 = _agg64(h, src, dst, z64)
    h2 = _mlp(h, agg2, W3, b3.reshape(1, H), W4, b4.reshape(1, H), H)
    psums, pcnt = _pool(h2, batch, z64[:G], z8[:G], jnp.ones((KP, 8), jnp.float32))

    return pl.pallas_call(
        _final_body,
        in_specs=[
            pl.BlockSpec((NC, G, H), lambda: (0, 0, 0)),
            pl.BlockSpec((NC, G, 8), lambda: (0, 0, 0)),
            pl.BlockSpec((H, 2), lambda: (0, 0)),
            pl.BlockSpec((1, 2), lambda: (0, 0)),
        ],
        out_specs=pl.BlockSpec((G, 2), lambda: (0, 0)),
        out_shape=jax.ShapeDtypeStruct((G, 2), jnp.float32),
    )(psums, pcnt, Wc, bc.reshape(1, 2))


# speedup vs baseline: 2.5358x; 1.0994x over previous
import functools

import jax
import jax.numpy as jnp
from jax import lax
from jax.experimental import pallas as pl
from jax.experimental.pallas import tpu as pltpu
from jax.experimental.pallas import tpu_sc as plsc

N = 50000
E = 1600000
H = 64
G = 512
D1 = 8
HALF = 25000
ACC = 25600
PT = ACC // 16
EPT = E // 16
WB = 64

NC, NS = 2, 16


def _sc_mesh():
    return plsc.VectorSubcoreMesh(
        core_axis_name="c", subcore_axis_name="s",
        num_cores=NC, num_subcores=NS)


def _make_agg(D, K, CPS):
    SUPE = K * CPS
    NSUP = EPT // SUPE

    @functools.partial(
        pl.kernel,
        out_type=jax.ShapeDtypeStruct((NC, ACC, D), jnp.float32),
        mesh=_sc_mesh(),
        compiler_params=pltpu.CompilerParams(use_tc_tiling_on_sc=False),
        scratch_types=[
            pltpu.VMEM((SUPE,), jnp.int32),
            pltpu.VMEM((SUPE,), jnp.int32),
            pltpu.VMEM((SUPE,), jnp.int32),
            pltpu.VMEM((SUPE,), jnp.int32),
            pltpu.VMEM((CPS, K), jnp.int32),
            pltpu.VMEM((CPS, K), jnp.int32),
            pltpu.VMEM((SUPE, D), jnp.float32),
            pltpu.VMEM((SUPE, D), jnp.float32),
            pltpu.VMEM((WB, D), jnp.float32),
            pltpu.VMEM_SHARED((ACC, D), jnp.float32),
            pltpu.SemaphoreType.DMA,
            pltpu.SemaphoreType.DMA,
            pltpu.SemaphoreType.DMA,
            pltpu.SemaphoreType.DMA,
            pltpu.SemaphoreType.DMA,
            pltpu.SemaphoreType.DMA,
        ],
    )
    def agg(table_hbm, src_hbm, dst_hbm, zeros_hbm, out_hbm,
            srcb0, srcb1, dstb0, dstb1, loc0, loc1,
            rows0, rows1, wb_v, acc,
            lsem0, lsem1, gsem0, gsem1, ssem0, ssem1):
        c = lax.axis_index("c")
        s = lax.axis_index("s")
        iota16 = lax.broadcasted_iota(jnp.int32, (16,), 0)
        srcb = (srcb0, srcb1)
        dstb = (dstb0, dstb1)
        locs = (loc0, loc1)
        rows = (rows0, rows1)
        lsem = (lsem0, lsem1)
        gsem = (gsem0, gsem1)
        ssem = (ssem0, ssem1)
        half0 = c * HALF
        ebase = s * EPT

        pltpu.sync_copy(zeros_hbm.at[pl.ds(s * PT, PT)],
                        acc.at[pl.ds(s * PT, PT)])
        plsc.subcore_barrier()

        def fire_loads(g, a):
            sb = ebase + g * SUPE
            pltpu.async_copy(src_hbm.at[pl.ds(sb, SUPE)], srcb[a], lsem[a])
            pltpu.async_copy(dst_hbm.at[pl.ds(sb, SUPE)], dstb[a], lsem[a])

        def wait_loads(a):
            pltpu.make_async_copy(src_hbm.at[pl.ds(0, SUPE)], srcb[a],
                                  lsem[a]).wait()
            pltpu.make_async_copy(dst_hbm.at[pl.ds(0, SUPE)], dstb[a],
                                  lsem[a]).wait()

        def compute_fire_gathers(a):
            for k in range(CPS):
                for t in range(K // 16):
                    o = k * K + t * 16
                    dv = dstb[a][pl.ds(o, 16)]
                    lv = dv - half0
                    inb = (lv >= 0) & (lv < HALF)
                    dummy = HALF + ((s * 16 + o + iota16) & 511)
                    locs[a][k, pl.ds(t * 16, 16)] = jnp.where(inb, lv, dummy)
                pltpu.async_copy(table_hbm.at[srcb[a].at[pl.ds(k * K, K)]],
                                 rows[a].at[pl.ds(k * K, K)], gsem[a])

        def wait_gathers_fire_scatters(a):
            for k in range(CPS):
                pltpu.make_async_copy(table_hbm.at[srcb[a].at[pl.ds(k * K, K)]],
                                      rows[a].at[pl.ds(k * K, K)],
                                      gsem[a]).wait()
                pltpu.async_copy(rows[a].at[pl.ds(k * K, K)],
                                 acc.at[locs[a].at[k]], ssem[a], add=True)

        def wait_scatters(a):
            for k in range(CPS):
                pltpu.make_async_copy(rows[a].at[pl.ds(k * K, K)],
                                      acc.at[locs[a].at[k]], ssem[a]).wait()

        def body(g, a, static):
            a1 = 1 - a
            wait_scatters(a)
            wait_gathers_fire_scatters(a1)
            if static:
                if g < NSUP - 1:
                    fire_loads(g + 1, a1)
            else:
                @pl.when(g < NSUP - 1)
                def _():
                    fire_loads(g + 1, a1)
            wait_loads(a)
            compute_fire_gathers(a)

        fire_loads(0, 0)
        fire_loads(1, 1)
        wait_loads(0)
        compute_fire_gathers(0)
        wait_gathers_fire_scatters(0)
        fire_loads(2, 0)
        wait_loads(1)
        compute_fire_gathers(1)
        body(2, 0, True)
        nxt = 3
        if (NSUP - nxt) % 2 == 1:
            body(3, 1, True)
            nxt = 4

        def pair(u, _):
            for p in range(2):
                g = nxt + 2 * u + p
                body(g, (nxt + p) % 2, False)
            return 0
        lax.fori_loop(0, (NSUP - nxt) // 2, pair, 0)

        last = (NSUP - 1) % 2
        wait_gathers_fire_scatters(last)
        wait_scatters(1 - last)
        wait_scatters(last)
        plsc.subcore_barrier()

        def wb_body(r, _):
            st = s * PT + r * WB
            pltpu.sync_copy(acc.at[pl.ds(st, WB)], wb_v)
            pltpu.sync_copy(wb_v, out_hbm.at[c, pl.ds(st, WB)])
            return 0
        lax.fori_loop(0, PT // WB, wb_body, 0)

    return agg


_agg8 = _make_agg(D1, 80, 10)
_agg64 = _make_agg(H, 80, 2)


KP = 80
_PCH = N // KP
_PPW = -(-_PCH // (NC * NS))
_GPT = G // NS


@functools.partial(
    pl.kernel,
    out_type=(jax.ShapeDtypeStruct((NC, G, H), jnp.float32),
              jax.ShapeDtypeStruct((NC, G, 8), jnp.float32)),
    mesh=_sc_mesh(),
    compiler_params=pltpu.CompilerParams(use_tc_tiling_on_sc=False),
    scratch_types=[
        pltpu.VMEM((KP,), jnp.int32),
        pltpu.VMEM((KP, H), jnp.float32),
        pltpu.VMEM((KP, 8), jnp.float32),
        pltpu.VMEM((_GPT, H), jnp.float32),
        pltpu.VMEM((_GPT, 8), jnp.float32),
        pltpu.VMEM_SHARED((G, H), jnp.float32),
        pltpu.VMEM_SHARED((G, 8), jnp.float32),
    ],
)
def _pool(h2_hbm, batch_hbm, zs_hbm, zc_hbm, ones_hbm, out_s, out_c,
          b_v, rows_v, ones_v, wbs_v, wbc_v, acc_s, acc_c):
    c = lax.axis_index("c")
    s = lax.axis_index("s")
    wid = s * NC + c

    pltpu.sync_copy(ones_hbm, ones_v)
    pltpu.sync_copy(zs_hbm.at[pl.ds(s * _GPT, _GPT)],
                    acc_s.at[pl.ds(s * _GPT, _GPT)])
    pltpu.sync_copy(zc_hbm.at[pl.ds(s * _GPT, _GPT)],
                    acc_c.at[pl.ds(s * _GPT, _GPT)])
    plsc.subcore_barrier()

    def body(i, _):
        ch = wid * _PPW + i

        @pl.when(ch < _PCH)
        def _():
            nb = ch * KP
            pltpu.sync_copy(batch_hbm.at[pl.ds(nb, KP)], b_v)
            pltpu.sync_copy(h2_hbm.at[pl.ds(nb, KP)], rows_v)
            pltpu.sync_copy(rows_v, acc_s.at[b_v], add=True)
            pltpu.sync_copy(ones_v, acc_c.at[b_v], add=True)
        return 0
    lax.fori_loop(0, _PPW, body, 0)
    plsc.subcore_barrier()

    pltpu.sync_copy(acc_s.at[pl.ds(s * _GPT, _GPT)], wbs_v)
    pltpu.sync_copy(wbs_v, out_s.at[c, pl.ds(s * _GPT, _GPT)])
    pltpu.sync_copy(acc_c.at[pl.ds(s * _GPT, _GPT)], wbc_v)
    pltpu.sync_copy(wbc_v, out_c.at[c, pl.ds(s * _GPT, _GPT)])


_R = 5000


def _mlp_body(x_ref, a_ref, w1_ref, b1_ref, w2_ref, b2_ref, o_ref):
    t = jnp.dot(x_ref[...] + a_ref[0], w1_ref[...],
                preferred_element_type=jnp.float32) + b1_ref[...]
    t = jnp.maximum(t, 0.0)
    h = jnp.dot(t, w2_ref[...], preferred_element_type=jnp.float32) + b2_ref[...]
    o_ref[...] = jnp.maximum(h, 0.0)


def _mlp(x, agg, w1, b1, w2, b2, din):
    return pl.pallas_call(
        _mlp_body,
        grid=(N // _R,),
        in_specs=[
            pl.BlockSpec((_R, din), lambda i: (i, 0)),
            pl.BlockSpec((1, _R, din), lambda i: (i // (HALF // _R),
                                                  i % (HALF // _R), 0)),
            pl.BlockSpec((din, H), lambda i: (0, 0)),
            pl.BlockSpec((1, H), lambda i: (0, 0)),
            pl.BlockSpec((H, H), lambda i: (0, 0)),
            pl.BlockSpec((1, H), lambda i: (0, 0)),
        ],
        out_specs=pl.BlockSpec((_R, H), lambda i: (i, 0)),
        out_shape=jax.ShapeDtypeStruct((N, H), jnp.float32),
    )(x, agg, w1, b1, w2, b2)


def _final_body(s_ref, c_ref, wc_ref, bc_ref, o_ref):
    sums = s_ref[0] + s_ref[1]
    cnt = jnp.maximum(c_ref[0][:, 0:1] + c_ref[1][:, 0:1], 1.0)
    pooled = sums / cnt
    o_ref[...] = jnp.dot(pooled, wc_ref[...],
                         preferred_element_type=jnp.float32) + bc_ref[...]


def kernel(x, edge_index, batch, W1, b1, W2, b2, W3, b3, W4, b4, Wc, bc):
    xp = jnp.pad(x, ((0, 0), (0, D1 - x.shape[1])))
    w1p = jnp.pad(W1, ((0, D1 - W1.shape[0]), (0, 0)))
    src = edge_index[0].astype(jnp.int32)
    dst = edge_index[1].astype(jnp.int32)
    batch = batch.astype(jnp.int32)
    z8 = jnp.zeros((ACC, D1), jnp.float32)
    z64 = jnp.zeros((ACC, H), jnp.float32)

    agg1 = _agg8(xp, src, dst, z8)
    h = _mlp(xp, agg1, w1p, b1.reshape(1, H), W2, b2.reshape(1, H), D1)
    agg2 = _agg64(h, src, dst, z64)
    h2 = _mlp(h, agg2, W3, b3.reshape(1, H), W4, b4.reshape(1, H), H)
    psums, pcnt = _pool(h2, batch, z64[:G], z8[:G], jnp.ones((KP, 8), jnp.float32))

    return pl.pallas_call(
        _final_body,
        in_specs=[
            pl.BlockSpec((NC, G, H), lambda: (0, 0, 0)),
            pl.BlockSpec((NC, G, 8), lambda: (0, 0, 0)),
            pl.BlockSpec((H, 2), lambda: (0, 0)),
            pl.BlockSpec((1, 2), lambda: (0, 0)),
        ],
        out_specs=pl.BlockSpec((G, 2), lambda: (0, 0)),
        out_shape=jax.ShapeDtypeStruct((G, 2), jnp.float32),
    )(psums, pcnt, Wc, bc.reshape(1, 2))
